# baseline (speedup 1.0000x reference)
import functools

import jax
import jax.numpy as jnp
from jax import lax
from jax.experimental import pallas as pl
from jax.experimental.pallas import tpu as pltpu
from jax.experimental.pallas import tpu_sc as plsc

HIDDEN = 128
SEQ = 20
NUM_CHARGES = 10
SPLIT = 1024
BBLK = 1024


def _sc_gather(table, idx, n):
    info = plsc.get_sparse_core_info()
    nc, ns = info.num_cores, info.num_subcores
    nw = nc * ns
    b_per_w = n // nw
    chunk = min(128, b_per_w)
    n_chunks = b_per_w // chunk
    mesh = plsc.VectorSubcoreMesh(core_axis_name="c", subcore_axis_name="s")

    @functools.partial(
        pl.kernel,
        mesh=mesh,
        out_type=jax.ShapeDtypeStruct((n, HIDDEN), jnp.float32),
        compiler_params=pltpu.CompilerParams(use_tc_tiling_on_sc=True),
        scratch_types=[
            pltpu.VMEM((b_per_w,), jnp.int32),
            pltpu.VMEM((b_per_w, HIDDEN), jnp.float32),
            pltpu.SemaphoreType.DMA,
        ],
    )
    def gather_kernel(table_hbm, idx_hbm, out_hbm, idx_v, rows_v, sem):
        wid = lax.axis_index("s") * nc + lax.axis_index("c")
        base = wid * b_per_w
        pltpu.sync_copy(idx_hbm.at[pl.ds(base, b_per_w)], idx_v)
        copies = []
        for j in range(n_chunks):
            copies.append(pltpu.async_copy(
                table_hbm.at[idx_v.at[pl.ds(j * chunk, chunk)]],
                rows_v.at[pl.ds(j * chunk, chunk)], sem))
        for c in copies:
            c.wait()
        pltpu.sync_copy(rows_v, out_hbm.at[pl.ds(base, b_per_w)])

    return gather_kernel(table, idx)


def _tc_upper(src_t, charge_hi_col, table):
    S, B, H = src_t.shape
    nb_hi = (B - SPLIT) // BBLK
    base = SPLIT // BBLK

    def body(src_ref, ch_ref, tab_ref, out_ref):
        ch = ch_ref[...].astype(jnp.int32)
        oh = (ch == lax.broadcasted_iota(jnp.int32, (BBLK, NUM_CHARGES), 1))
        emb = jnp.dot(oh.astype(jnp.float32), tab_ref[...],
                      preferred_element_type=jnp.float32)
        out_ref[...] = src_ref[...] + emb[None, :, :]

    return pl.pallas_call(
        body,
        grid=(nb_hi,),
        in_specs=[
            pl.BlockSpec((S, BBLK, H), lambda i: (0, base + i, 0)),
            pl.BlockSpec((BBLK, 1), lambda i: (i, 0)),
            pl.BlockSpec((NUM_CHARGES, H), lambda i: (0, 0)),
        ],
        out_specs=pl.BlockSpec((S, BBLK, H), lambda i: (0, base + i, 0)),
        out_shape=jax.ShapeDtypeStruct((S, B, H), src_t.dtype),
    )(src_t, charge_hi_col, table)


def _tc_lower(src_t, emb_lo, prev):
    S, B, H = src_t.shape
    nb_lo = SPLIT // BBLK

    def body(src_ref, emb_ref, prev_ref, out_ref):
        del prev_ref
        out_ref[...] = src_ref[...] + emb_ref[...][None, :, :]

    return pl.pallas_call(
        body,
        grid=(nb_lo,),
        in_specs=[
            pl.BlockSpec((S, BBLK, H), lambda i: (0, i, 0)),
            pl.BlockSpec((BBLK, H), lambda i: (i, 0)),
            pl.BlockSpec(memory_space=pltpu.MemorySpace.HBM),
        ],
        out_specs=pl.BlockSpec((S, BBLK, H), lambda i: (0, i, 0)),
        out_shape=jax.ShapeDtypeStruct((S, B, H), src_t.dtype),
        input_output_aliases={2: 0},
    )(src_t, emb_lo, prev)


def kernel(src, charge, charge_table):
    charge32 = charge.astype(jnp.int32)
    emb_lo = _sc_gather(charge_table, charge32, SPLIT)
    src_t = jnp.transpose(src, (1, 0, 2))
    ch_hi = charge32[SPLIT:].reshape(-1, 1)
    partial_t = _tc_upper(src_t, ch_hi, charge_table)
    out_t = _tc_lower(src_t, emb_lo, partial_t)
    return jnp.transpose(out_t, (1, 0, 2))


# speedup vs baseline: 1.0367x; 1.0367x over previous
import functools

import jax
import jax.numpy as jnp
from jax import lax
from jax.experimental import pallas as pl
from jax.experimental.pallas import tpu as pltpu
from jax.experimental.pallas import tpu_sc as plsc

HIDDEN = 128
SEQ = 20
NUM_CHARGES = 10
BBLK = 1024
SPLIT = BBLK


def _sc_gather(table, idx, n):
    info = plsc.get_sparse_core_info()
    nc, ns = info.num_cores, info.num_subcores
    nw = nc * ns
    b_per_w = n // nw
    chunk = min(128, b_per_w)
    n_chunks = b_per_w // chunk
    mesh = plsc.VectorSubcoreMesh(core_axis_name="c", subcore_axis_name="s")

    @functools.partial(
        pl.kernel,
        mesh=mesh,
        out_type=jax.ShapeDtypeStruct((n, HIDDEN), jnp.float32),
        compiler_params=pltpu.CompilerParams(use_tc_tiling_on_sc=True),
        scratch_types=[
            pltpu.VMEM((b_per_w,), jnp.int32),
            pltpu.VMEM((b_per_w, HIDDEN), jnp.float32),
            pltpu.SemaphoreType.DMA,
        ],
    )
    def gather_kernel(table_hbm, idx_hbm, out_hbm, idx_v, rows_v, sem):
        wid = lax.axis_index("s") * nc + lax.axis_index("c")
        base = wid * b_per_w
        pltpu.sync_copy(idx_hbm.at[pl.ds(base, b_per_w)], idx_v)
        copies = []
        for j in range(n_chunks):
            copies.append(pltpu.async_copy(
                table_hbm.at[idx_v.at[pl.ds(j * chunk, chunk)]],
                rows_v.at[pl.ds(j * chunk, chunk)], sem))
        for c in copies:
            c.wait()
        pltpu.sync_copy(rows_v, out_hbm.at[pl.ds(base, b_per_w)])

    return gather_kernel(table, idx)


def _tc_add(src_t, emb_lo, ch_col, table):
    S, B, H = src_t.shape
    nb = B // BBLK

    def body(src_ref, emb_ref, ch_ref, tab_ref, out_ref):
        i = pl.program_id(0)

        @pl.when(i == 0)
        def _():
            out_ref[...] = src_ref[...] + emb_ref[...][None, :, :]

        @pl.when(i > 0)
        def _():
            ch = ch_ref[...].astype(jnp.int32)
            oh = (ch == lax.broadcasted_iota(jnp.int32, (BBLK, NUM_CHARGES), 1))
            emb = jnp.dot(oh.astype(jnp.float32), tab_ref[...],
                          preferred_element_type=jnp.float32)
            out_ref[...] = src_ref[...] + emb[None, :, :]

    return pl.pallas_call(
        body,
        grid=(nb,),
        in_specs=[
            pl.BlockSpec((S, BBLK, H), lambda i: (0, i, 0)),
            pl.BlockSpec((SPLIT, H), lambda i: (0, 0)),
            pl.BlockSpec((BBLK, 1), lambda i: (i, 0)),
            pl.BlockSpec((NUM_CHARGES, H), lambda i: (0, 0)),
        ],
        out_specs=pl.BlockSpec((S, BBLK, H), lambda i: (0, i, 0)),
        out_shape=jax.ShapeDtypeStruct((S, B, H), src_t.dtype),
    )(src_t, emb_lo, ch_col, table)


def kernel(src, charge, charge_table):
    charge32 = charge.astype(jnp.int32)
    emb_lo = _sc_gather(charge_table, charge32, SPLIT)
    src_t = jnp.transpose(src, (1, 0, 2))
    ch_col = charge32.astype(jnp.int8).reshape(-1, 1)
    out_t = _tc_add(src_t, emb_lo, ch_col, charge_table)
    return jnp.transpose(out_t, (1, 0, 2))
